# baseline (speedup 1.0000x reference)
import functools
import math

import jax
import jax.numpy as jnp
from jax import lax
from jax.experimental import pallas as pl
from jax.experimental.pallas import tpu as pltpu
from jax.experimental.pallas import tpu_sc as plsc

_SMOOTHING = 0.1
_CONF = 1.0 - _SMOOTHING

_NC = 2
_NS = 16
_NW = _NC * _NS
_L = 16

_CLS_PER_BLOCK = 2000


def _sc_gather_body(xt_hbm, tgt_hbm, tiles_hbm, tgt_v, tiles_v, sem,
                    *, rows_per_worker):
    wid = lax.axis_index("s") * _NC + lax.axis_index("c")
    base = wid * rows_per_worker
    pltpu.sync_copy(tgt_hbm.at[pl.ds(base, rows_per_worker)], tgt_v)
    copies = []
    for r in range(rows_per_worker):
        if r % _L == 0:
            tvec = tgt_v[pl.ds(r, _L)]
        c0 = pl.multiple_of(jnp.bitwise_and(tvec[r % _L], ~7), 8)
        b0 = pl.multiple_of(jnp.bitwise_and(base + r, ~127), 128)
        copies.append(pltpu.async_copy(
            xt_hbm.at[pl.ds(c0, 8), pl.ds(b0, 128)], tiles_v.at[r], sem))
    for cp in copies:
        cp.wait()
    pltpu.sync_copy(tiles_v, tiles_hbm.at[pl.ds(base, rows_per_worker)])


def _make_sc_gather(batch, num_classes):
    rows_per_worker = batch // _NW
    body = functools.partial(_sc_gather_body, rows_per_worker=rows_per_worker)
    return pl.kernel(
        body,
        out_type=jax.ShapeDtypeStruct((batch, 8, 128), jnp.float32),
        mesh=plsc.VectorSubcoreMesh(core_axis_name="c", subcore_axis_name="s"),
        scratch_types=[
            pltpu.VMEM((rows_per_worker,), jnp.int32),
            pltpu.VMEM((rows_per_worker, 8, 128), jnp.float32),
            pltpu.SemaphoreType.DMA,
        ],
    )


def _tc_loss_body(x_ref, tiles_ref, tgt_ref, out_ref, m_ref, se_ref, sx_ref,
                  *, coef_l, coef_s, coef_x, const_total, batch):
    i = pl.program_id(0)

    @pl.when(i == 0)
    def _init():
        m_ref[...] = jnp.full_like(m_ref[...], -jnp.inf)
        se_ref[...] = jnp.zeros_like(se_ref[...])
        sx_ref[...] = jnp.zeros_like(sx_ref[...])

    x = x_ref[...]
    m_old = m_ref[...]
    m_new = jnp.maximum(m_old, jnp.max(x, axis=0, keepdims=True))
    se_ref[...] = (se_ref[...] * jnp.exp(m_old - m_new)
                   + jnp.sum(jnp.exp(x - m_new), axis=0, keepdims=True))
    sx_ref[...] += jnp.sum(x, axis=0, keepdims=True)
    m_ref[...] = m_new

    @pl.when(i == pl.num_programs(0) - 1)
    def _fini():
        lse = m_ref[...] + jnp.log(se_ref[...])
        partial = coef_l * jnp.sum(lse) - coef_s * jnp.sum(sx_ref[...])
        t = tgt_ref[...].reshape(batch, 1, 1)
        b_ids = lax.broadcasted_iota(jnp.int32, (batch, 1, 1), 0)
        sub = lax.broadcasted_iota(jnp.int32, (batch, 8, 128), 1)
        lane = lax.broadcasted_iota(jnp.int32, (batch, 8, 128), 2)
        mask = ((sub == jnp.bitwise_and(t, 7))
                & (lane == jnp.bitwise_and(b_ids, 127)))
        xt_sum = jnp.sum(jnp.where(mask, tiles_ref[...], 0.0))
        val = (const_total + partial - coef_x * xt_sum) / batch
        out_ref[...] = jnp.full((1, 1), val, dtype=jnp.float32)


def kernel(output, target):
    batch, num_classes = output.shape
    s = _SMOOTHING / (num_classes - 1)
    coef_l = s * num_classes + (_CONF - s)
    coef_x = _CONF - s
    const_total = batch * ((num_classes - 1) * s * math.log(s)
                           + _CONF * math.log(_CONF))

    xt = output.T
    tgt32 = target.astype(jnp.int32)
    tiles = _make_sc_gather(batch, num_classes)(xt, tgt32)

    n_blocks = num_classes // _CLS_PER_BLOCK
    body = functools.partial(_tc_loss_body, coef_l=coef_l, coef_s=s,
                             coef_x=coef_x, const_total=const_total,
                             batch=batch)
    res = pl.pallas_call(
        body,
        grid=(n_blocks,),
        in_specs=[
            pl.BlockSpec((_CLS_PER_BLOCK, batch), lambda i: (i, 0)),
            pl.BlockSpec((batch, 8, 128), lambda i: (0, 0, 0)),
            pl.BlockSpec((batch, 1), lambda i: (0, 0)),
        ],
        out_specs=pl.BlockSpec((1, 1), lambda i: (0, 0)),
        out_shape=jax.ShapeDtypeStruct((1, 1), jnp.float32),
        scratch_shapes=[
            pltpu.VMEM((1, batch), jnp.float32),
            pltpu.VMEM((1, batch), jnp.float32),
            pltpu.VMEM((1, batch), jnp.float32),
        ],
    )(xt, tiles, tgt32.reshape(batch, 1))
    return res[0, 0]


# speedup vs baseline: 1.0898x; 1.0898x over previous
import functools
import math

import jax
import jax.numpy as jnp
from jax import lax
from jax.experimental import pallas as pl
from jax.experimental.pallas import tpu as pltpu
from jax.experimental.pallas import tpu_sc as plsc

_SMOOTHING = 0.1
_CONF = 1.0 - _SMOOTHING

_NC = 2
_NS = 16
_NW = _NC * _NS
_L = 16

_CLS_PER_BLOCK = 5000


def _sc_gather_body(xt_hbm, tgt_hbm, tiles_hbm, tgt_v, tiles_v, sem,
                    *, rows_per_worker):
    wid = lax.axis_index("s") * _NC + lax.axis_index("c")
    base = wid * rows_per_worker
    pltpu.sync_copy(tgt_hbm.at[pl.ds(base, rows_per_worker)], tgt_v)
    copies = []
    for r in range(rows_per_worker):
        if r % _L == 0:
            tvec = tgt_v[pl.ds(r, _L)]
        c0 = pl.multiple_of(jnp.bitwise_and(tvec[r % _L], ~7), 8)
        b0 = pl.multiple_of(jnp.bitwise_and(base + r, ~127), 128)
        copies.append(pltpu.async_copy(
            xt_hbm.at[pl.ds(c0, 8), pl.ds(b0, 128)], tiles_v.at[r], sem))
    for cp in copies:
        cp.wait()
    pltpu.sync_copy(tiles_v, tiles_hbm.at[pl.ds(base, rows_per_worker)])


def _make_sc_gather(batch, num_classes):
    rows_per_worker = batch // _NW
    body = functools.partial(_sc_gather_body, rows_per_worker=rows_per_worker)
    return pl.kernel(
        body,
        out_type=jax.ShapeDtypeStruct((batch, 8, 128), jnp.float32),
        mesh=plsc.VectorSubcoreMesh(core_axis_name="c", subcore_axis_name="s"),
        scratch_types=[
            pltpu.VMEM((rows_per_worker,), jnp.int32),
            pltpu.VMEM((rows_per_worker, 8, 128), jnp.float32),
            pltpu.SemaphoreType.DMA,
        ],
    )


def _tc_loss_body(x_ref, tiles_ref, tgt_ref, out_ref, m_ref, se_ref, sx_ref,
                  *, coef_l, coef_s, coef_x, const_total, batch):
    i = pl.program_id(0)

    @pl.when(i == 0)
    def _init():
        m_ref[...] = jnp.full_like(m_ref[...], -jnp.inf)
        se_ref[...] = jnp.zeros_like(se_ref[...])
        sx_ref[...] = jnp.zeros_like(sx_ref[...])

    x = x_ref[...]
    m_old = m_ref[...]
    m_new = jnp.maximum(m_old, jnp.max(x, axis=0, keepdims=True))
    se_ref[...] = (se_ref[...] * jnp.exp(m_old - m_new)
                   + jnp.sum(jnp.exp(x - m_new), axis=0, keepdims=True))
    sx_ref[...] += jnp.sum(x, axis=0, keepdims=True)
    m_ref[...] = m_new

    @pl.when(i == pl.num_programs(0) - 1)
    def _fini():
        lse = m_ref[...] + jnp.log(se_ref[...])
        partial = coef_l * jnp.sum(lse) - coef_s * jnp.sum(sx_ref[...])
        t = tgt_ref[...].reshape(batch, 1, 1)
        b_ids = lax.broadcasted_iota(jnp.int32, (batch, 1, 1), 0)
        sub = lax.broadcasted_iota(jnp.int32, (batch, 8, 128), 1)
        lane = lax.broadcasted_iota(jnp.int32, (batch, 8, 128), 2)
        mask = ((sub == jnp.bitwise_and(t, 7))
                & (lane == jnp.bitwise_and(b_ids, 127)))
        xt_sum = jnp.sum(jnp.where(mask, tiles_ref[...], 0.0))
        val = (const_total + partial - coef_x * xt_sum) / batch
        out_ref[...] = jnp.full((1, 1), val, dtype=jnp.float32)


def kernel(output, target):
    batch, num_classes = output.shape
    s = _SMOOTHING / (num_classes - 1)
    coef_l = s * num_classes + (_CONF - s)
    coef_x = _CONF - s
    const_total = batch * ((num_classes - 1) * s * math.log(s)
                           + _CONF * math.log(_CONF))

    xt = output.T
    tgt32 = target.astype(jnp.int32)
    tiles = _make_sc_gather(batch, num_classes)(xt, tgt32)

    n_blocks = num_classes // _CLS_PER_BLOCK
    body = functools.partial(_tc_loss_body, coef_l=coef_l, coef_s=s,
                             coef_x=coef_x, const_total=const_total,
                             batch=batch)
    res = pl.pallas_call(
        body,
        grid=(n_blocks,),
        in_specs=[
            pl.BlockSpec((_CLS_PER_BLOCK, batch), lambda i: (i, 0)),
            pl.BlockSpec((batch, 8, 128), lambda i: (0, 0, 0)),
            pl.BlockSpec((batch, 1), lambda i: (0, 0)),
        ],
        out_specs=pl.BlockSpec((1, 1), lambda i: (0, 0)),
        out_shape=jax.ShapeDtypeStruct((1, 1), jnp.float32),
        compiler_params=pltpu.CompilerParams(vmem_limit_bytes=100 * 1024 * 1024),
        scratch_shapes=[
            pltpu.VMEM((1, batch), jnp.float32),
            pltpu.VMEM((1, batch), jnp.float32),
            pltpu.VMEM((1, batch), jnp.float32),
        ],
    )(xt, tiles, tgt32.reshape(batch, 1))
    return res[0, 0]
